# baseline (speedup 1.0000x reference)
import functools

import jax
import jax.numpy as jnp
from jax import lax
from jax.experimental import pallas as pl
from jax.experimental.pallas import tpu as pltpu
from jax.experimental.pallas import tpu_sc as plsc

BATCH = 16384
HIST = 200
EMBED = 64
NCLS = 100

NC = 2
NS = 16
NW = NC * NS
S_PER_W = BATCH // NW
SB = 64
NBLK = S_PER_W // SB
NBUF = 8
LOOK = 7
CH0 = 104
CH1 = HIST - CH0
RU = 8
LANES = 16
NSEG = EMBED // LANES

_mesh = plsc.VectorSubcoreMesh(core_axis_name="c", subcore_axis_name="s")


@functools.partial(
    pl.kernel,
    mesh=_mesh,
    out_type=jax.ShapeDtypeStruct((BATCH, EMBED), jnp.float32),
    scratch_types=[
        pltpu.VMEM((SB * HIST,), jnp.int32),
        pltpu.VMEM((NBUF, HIST, EMBED), jnp.float32),
        pltpu.VMEM((SB, EMBED), jnp.float32),
        [pltpu.SemaphoreType.DMA] * NBUF,
    ],
    compiler_params=pltpu.CompilerParams(use_tc_tiling_on_sc=False),
)
def _pool_kernel(x_hbm, table_hbm, pooled_hbm, idx_v, rows_v, pooled_v, sems):
    wid = lax.axis_index("s") * NC + lax.axis_index("c")
    base = wid * S_PER_W

    def fire(s_local, buf):
        off = pl.multiple_of(s_local * HIST, 8)
        pltpu.async_copy(
            table_hbm.at[idx_v.at[pl.ds(off, CH0)]],
            rows_v.at[buf, pl.ds(0, CH0)], sems[buf])
        off1 = pl.multiple_of(s_local * HIST + CH0, 8)
        pltpu.async_copy(
            table_hbm.at[idx_v.at[pl.ds(off1, CH1)]],
            rows_v.at[buf, pl.ds(CH0, CH1)], sems[buf])

    def drain(buf):
        pltpu.make_async_copy(
            table_hbm.at[pl.ds(0, HIST)], rows_v.at[buf], sems[buf]).wait()

    def accumulate(buf, s_local):
        zero = jnp.zeros((LANES,), jnp.float32)

        def body(r, acc):
            acc = list(acc)
            for rr in range(RU):
                row = r * RU + rr
                half = (rr % 2) * NSEG
                for d in range(NSEG):
                    acc[half + d] = acc[half + d] + rows_v[
                        buf, row, pl.ds(d * LANES, LANES)]
            return tuple(acc)

        acc = lax.fori_loop(0, HIST // RU, body, (zero,) * (2 * NSEG))
        for d in range(NSEG):
            pooled_v[s_local, pl.ds(d * LANES, LANES)] = acc[d] + acc[NSEG + d]

    def block_body(blk, carry):
        row0 = base + blk * SB
        pltpu.sync_copy(x_hbm.at[pl.ds(row0 * HIST, SB * HIST)], idx_v)
        for u in range(LOOK):
            fire(u, u)

        def octet(it, c):
            s0 = it * NBUF
            for u in range(NBUF):
                s = s0 + u
                drain(u)

                @pl.when(s + LOOK < SB)
                def _():
                    fire(s + LOOK, (u + LOOK) % NBUF)

                accumulate(u, s)
            return c

        lax.fori_loop(0, SB // NBUF, octet, 0)
        pltpu.sync_copy(pooled_v, pooled_hbm.at[pl.ds(row0, SB)])
        return carry

    lax.fori_loop(0, NBLK, block_body, 0)


def _cls_body(p_ref, w_ref, b_ref, o_ref):
    o_ref[...] = lax.dot_general(
        p_ref[...], w_ref[...], (((1,), (1,)), ((), ())),
        preferred_element_type=jnp.float32) * (1.0 / HIST) + b_ref[...]


_BM = 2048


def kernel(x, table, W, b):
    x_flat = x.astype(jnp.int32).reshape(BATCH * HIST)
    pooled = _pool_kernel(x_flat, table)
    out = pl.pallas_call(
        _cls_body,
        grid=(BATCH // _BM,),
        in_specs=[
            pl.BlockSpec((_BM, EMBED), lambda i: (i, 0)),
            pl.BlockSpec((NCLS, EMBED), lambda i: (0, 0)),
            pl.BlockSpec((1, NCLS), lambda i: (0, 0)),
        ],
        out_specs=pl.BlockSpec((_BM, NCLS), lambda i: (i, 0)),
        out_shape=jax.ShapeDtypeStruct((BATCH, NCLS), jnp.float32),
    )(pooled, W, b.reshape(1, NCLS))
    return out


# speedup vs baseline: 1.0026x; 1.0026x over previous
import functools

import jax
import jax.numpy as jnp
from jax import lax
from jax.experimental import pallas as pl
from jax.experimental.pallas import tpu as pltpu
from jax.experimental.pallas import tpu_sc as plsc

BATCH = 16384
HIST = 200
EMBED = 64
NCLS = 100

NC = 2
NS = 16
NW = NC * NS
S_PER_W = BATCH // NW
SB = 64
NBLK = S_PER_W // SB
NBUF = 8
LOOK = 6
CH0 = 128
CH1 = HIST - CH0
RU = 8
LANES = 16
NSEG = EMBED // LANES

_mesh = plsc.VectorSubcoreMesh(core_axis_name="c", subcore_axis_name="s")


@functools.partial(
    pl.kernel,
    mesh=_mesh,
    out_type=jax.ShapeDtypeStruct((BATCH, EMBED), jnp.float32),
    scratch_types=[
        pltpu.VMEM((SB * HIST,), jnp.int32),
        pltpu.VMEM((NBUF, HIST, EMBED), jnp.float32),
        pltpu.VMEM((SB, EMBED), jnp.float32),
        [pltpu.SemaphoreType.DMA] * NBUF,
    ],
    compiler_params=pltpu.CompilerParams(use_tc_tiling_on_sc=False),
)
def _pool_kernel(x_hbm, table_hbm, pooled_hbm, idx_v, rows_v, pooled_v, sems):
    wid = lax.axis_index("s") * NC + lax.axis_index("c")
    base = wid * S_PER_W

    def fire(s_local, buf):
        off = pl.multiple_of(s_local * HIST, 8)
        pltpu.async_copy(
            table_hbm.at[idx_v.at[pl.ds(off, CH0)]],
            rows_v.at[buf, pl.ds(0, CH0)], sems[buf])
        off1 = pl.multiple_of(s_local * HIST + CH0, 8)
        pltpu.async_copy(
            table_hbm.at[idx_v.at[pl.ds(off1, CH1)]],
            rows_v.at[buf, pl.ds(CH0, CH1)], sems[buf])

    def drain(buf):
        pltpu.make_async_copy(
            table_hbm.at[pl.ds(0, HIST)], rows_v.at[buf], sems[buf]).wait()

    def accumulate(buf, s_local):
        zero = jnp.zeros((LANES,), jnp.float32)

        def body(r, acc):
            acc = list(acc)
            for rr in range(RU):
                row = r * RU + rr
                half = (rr % 2) * NSEG
                for d in range(NSEG):
                    acc[half + d] = acc[half + d] + rows_v[
                        buf, row, pl.ds(d * LANES, LANES)]
            return tuple(acc)

        acc = lax.fori_loop(0, HIST // RU, body, (zero,) * (2 * NSEG))
        for d in range(NSEG):
            pooled_v[s_local, pl.ds(d * LANES, LANES)] = acc[d] + acc[NSEG + d]

    def block_body(blk, carry):
        row0 = base + blk * SB
        pltpu.sync_copy(x_hbm.at[pl.ds(row0 * HIST, SB * HIST)], idx_v)
        for u in range(LOOK):
            fire(u, u)

        def octet(it, c):
            s0 = it * NBUF
            for u in range(NBUF):
                s = s0 + u
                drain(u)

                @pl.when(s + LOOK < SB)
                def _():
                    fire(s + LOOK, (u + LOOK) % NBUF)

                accumulate(u, s)
            return c

        lax.fori_loop(0, SB // NBUF, octet, 0)
        pltpu.sync_copy(pooled_v, pooled_hbm.at[pl.ds(row0, SB)])
        return carry

    lax.fori_loop(0, NBLK, block_body, 0)


def _cls_body(p_ref, w_ref, b_ref, o_ref):
    o_ref[...] = lax.dot_general(
        p_ref[...], w_ref[...], (((1,), (1,)), ((), ())),
        preferred_element_type=jnp.float32) * (1.0 / HIST) + b_ref[...]


_BM = 2048


def kernel(x, table, W, b):
    x_flat = x.astype(jnp.int32).reshape(BATCH * HIST)
    pooled = _pool_kernel(x_flat, table)
    out = pl.pallas_call(
        _cls_body,
        grid=(BATCH // _BM,),
        in_specs=[
            pl.BlockSpec((_BM, EMBED), lambda i: (i, 0)),
            pl.BlockSpec((NCLS, EMBED), lambda i: (0, 0)),
            pl.BlockSpec((1, NCLS), lambda i: (0, 0)),
        ],
        out_specs=pl.BlockSpec((_BM, NCLS), lambda i: (i, 0)),
        out_shape=jax.ShapeDtypeStruct((BATCH, NCLS), jnp.float32),
    )(pooled, W, b.reshape(1, NCLS))
    return out
